# baseline (speedup 1.0000x reference)
import functools
import jax
import jax.numpy as jnp
from jax import lax
from jax.experimental import pallas as pl
from jax.experimental.pallas import tpu as pltpu
from jax.experimental.pallas import tpu_sc as plsc

F = 26
V = 100000
E = 16
B = 4096
D1 = 32
D2 = 32

NC = 2
NS = 16
NW = NC * NS
N = B * F
NPW = N // NW

CH = 128
NCHUNK = NPW // CH
NROW = N // CH


@functools.lru_cache(maxsize=None)
def _get_sc_gather():
    mesh = plsc.VectorSubcoreMesh(
        core_axis_name="c", subcore_axis_name="s",
        num_cores=NC, num_subcores=NS)

    @functools.partial(
        pl.kernel,
        out_type=(
            jax.ShapeDtypeStruct((NROW, CH, E), jnp.float32),
            jax.ShapeDtypeStruct((NROW, CH), jnp.float32),
        ),
        mesh=mesh,
        scratch_types=[
            pltpu.VMEM((NCHUNK, CH), jnp.int32),
            pltpu.VMEM((NCHUNK, CH, E), jnp.float32),
            pltpu.VMEM((NCHUNK, CH), jnp.float32),
            pltpu.SemaphoreType.DMA,
            pltpu.SemaphoreType.DMA,
        ],
        compiler_params=pltpu.CompilerParams(use_tc_tiling_on_sc=False),
    )
    def _sc_gather(idx_hbm, fm2_hbm, fm1_hbm, e2_out, e1_out,
                   idx_v, rows_v, r1_v, sem2, sem1):
        wid = lax.axis_index("s") * NC + lax.axis_index("c")
        base = wid * NCHUNK
        pltpu.sync_copy(idx_hbm.at[pl.ds(base, NCHUNK)], idx_v)
        cps = []
        for j in range(NCHUNK):
            cps.append(pltpu.async_copy(
                fm2_hbm.at[idx_v.at[j]], rows_v.at[j], sem2))
            cps.append(pltpu.async_copy(
                fm1_hbm.at[idx_v.at[j]], r1_v.at[j], sem1))
        for cp in cps:
            cp.wait()
        pltpu.sync_copy(rows_v, e2_out.at[pl.ds(base, NCHUNK)])
        pltpu.sync_copy(r1_v, e1_out.at[pl.ds(base, NCHUNK)])

    return _sc_gather


BB = 512


def _tc_body(e2_ref, e1_ref, xv_ref, rep_ref, ssum_ref, w1_ref, b1_ref,
             w2_ref, b2_ref, bias_ref, out_ref):
    e2 = e2_ref[...]
    xv = xv_ref[...]
    xvr = jnp.dot(xv, rep_ref[...], preferred_element_type=jnp.float32)
    e2v = e2 * xvr
    s = jnp.dot(e2v, ssum_ref[...], preferred_element_type=jnp.float32)
    sq = jnp.dot(e2v * e2v, ssum_ref[...], preferred_element_type=jnp.float32)
    second = 0.5 * jnp.sum(s * s - sq, axis=1)
    first = jnp.sum(e1_ref[...] * xv, axis=1)
    x1 = jnp.maximum(
        jnp.dot(e2v, w1_ref[...], preferred_element_type=jnp.float32)
        + b1_ref[...], 0.0)
    x2 = jnp.maximum(
        jnp.dot(x1, w2_ref[...], preferred_element_type=jnp.float32)
        + b2_ref[...], 0.0)
    out_ref[...] = first + second + jnp.sum(x2, axis=1) + bias_ref[0, 0]


@jax.jit
def _deepfm(Xi, Xv, fm1, fm2, bias, W1, b1, W2, b2):
    idx = Xi[:, :, 0]
    flat_idx = (idx + jnp.arange(F, dtype=jnp.int32)[None, :] * V).reshape(
        NROW, CH)
    fm2_flat = fm2.reshape(F * V, E)
    fm1_flat = fm1.reshape(F * V)

    e2, e1 = _get_sc_gather()(flat_idx, fm2_flat, fm1_flat)
    e2 = e2.reshape(B, F * E)
    e1 = e1.reshape(B, F)

    eye = jnp.eye(E, dtype=jnp.float32)
    rep = (jnp.arange(F)[:, None] ==
           (jnp.arange(F * E)[None, :] // E)).astype(jnp.float32)
    ssum = jnp.tile(eye, (F, 1))

    grid = (B // BB,)
    out = pl.pallas_call(
        _tc_body,
        grid=grid,
        in_specs=[
            pl.BlockSpec((BB, F * E), lambda i: (i, 0)),
            pl.BlockSpec((BB, F), lambda i: (i, 0)),
            pl.BlockSpec((BB, F), lambda i: (i, 0)),
            pl.BlockSpec((F, F * E), lambda i: (0, 0)),
            pl.BlockSpec((F * E, E), lambda i: (0, 0)),
            pl.BlockSpec((F * E, D1), lambda i: (0, 0)),
            pl.BlockSpec((1, D1), lambda i: (0, 0)),
            pl.BlockSpec((D1, D2), lambda i: (0, 0)),
            pl.BlockSpec((1, D2), lambda i: (0, 0)),
            pl.BlockSpec((1, 1), lambda i: (0, 0)),
        ],
        out_specs=pl.BlockSpec((BB,), lambda i: (i,)),
        out_shape=jax.ShapeDtypeStruct((B,), jnp.float32),
    )(e2, e1, Xv, rep, ssum, W1, b1.reshape(1, D1), W2, b2.reshape(1, D2),
      bias.reshape(1, 1))
    return out


def kernel(Xi, Xv, fm1, fm2, bias, W1, b1, W2, b2):
    return _deepfm(Xi, Xv, fm1, fm2, bias, W1, b1, W2, b2)


# speedup vs baseline: 1.0005x; 1.0005x over previous
import functools
import jax
import jax.numpy as jnp
from jax import lax
from jax.experimental import pallas as pl
from jax.experimental.pallas import tpu as pltpu
from jax.experimental.pallas import tpu_sc as plsc

F = 26
V = 100000
E = 16
B = 4096
D1 = 32
D2 = 32

NC = 2
NS = 16
NW = NC * NS
N = B * F
NPW = N // NW

CH = 128
NCHUNK = NPW // CH
NROW = N // CH


@functools.lru_cache(maxsize=None)
def _get_sc_gather():
    mesh = plsc.VectorSubcoreMesh(
        core_axis_name="c", subcore_axis_name="s",
        num_cores=NC, num_subcores=NS)

    @functools.partial(
        pl.kernel,
        out_type=(
            jax.ShapeDtypeStruct((NROW, CH, E), jnp.float32),
            jax.ShapeDtypeStruct((NROW, CH), jnp.float32),
        ),
        mesh=mesh,
        scratch_types=[
            pltpu.VMEM((NCHUNK, CH), jnp.int32),
            pltpu.VMEM((NCHUNK, CH, E), jnp.float32),
            pltpu.VMEM((NCHUNK, CH), jnp.float32),
            pltpu.SemaphoreType.DMA,
            pltpu.SemaphoreType.DMA,
        ],
        compiler_params=pltpu.CompilerParams(use_tc_tiling_on_sc=False),
    )
    def _sc_gather(idx_hbm, fm2_hbm, fm1_hbm, e2_out, e1_out,
                   idx_v, rows_v, r1_v, sem2, sem1):
        wid = lax.axis_index("s") * NC + lax.axis_index("c")
        base = wid * NCHUNK
        pltpu.sync_copy(idx_hbm.at[pl.ds(base, NCHUNK)], idx_v)
        cps = []
        for j in range(NCHUNK):
            cps.append(pltpu.async_copy(
                fm2_hbm.at[idx_v.at[j]], rows_v.at[j], sem2))
            cps.append(pltpu.async_copy(
                fm1_hbm.at[idx_v.at[j]], r1_v.at[j], sem1))
        for cp in cps:
            cp.wait()
        pltpu.sync_copy(rows_v, e2_out.at[pl.ds(base, NCHUNK)])
        pltpu.sync_copy(r1_v, e1_out.at[pl.ds(base, NCHUNK)])

    return _sc_gather


BB = 512


def _tc_body(e2_ref, e1_ref, xv_ref, rep_ref, ssum_ref, w1_ref, b1_ref,
             w2_ref, b2_ref, bias_ref, out_ref):
    e2 = e2_ref[...]
    xv = xv_ref[...]
    xvr = jnp.dot(xv, rep_ref[...], preferred_element_type=jnp.float32)
    e2v = e2 * xvr
    s = jnp.dot(e2v, ssum_ref[...], preferred_element_type=jnp.float32)
    sq = jnp.dot(e2v * e2v, ssum_ref[...], preferred_element_type=jnp.float32)
    second = 0.5 * jnp.sum(s * s - sq, axis=1)
    first = jnp.sum(e1_ref[...] * xv, axis=1)
    x1 = jnp.maximum(
        jnp.dot(e2v, w1_ref[...], preferred_element_type=jnp.float32)
        + b1_ref[...], 0.0)
    x2 = jnp.maximum(
        jnp.dot(x1, w2_ref[...], preferred_element_type=jnp.float32)
        + b2_ref[...], 0.0)
    out_ref[...] = first + second + jnp.sum(x2, axis=1) + bias_ref[0, 0]


@jax.jit
def _deepfm(Xi, Xv, fm1, fm2, bias, W1, b1, W2, b2):
    idx = Xi[:, :, 0]
    flat_idx = (idx + jnp.arange(F, dtype=jnp.int32)[None, :] * V).reshape(
        NROW, CH)
    fm2_flat = lax.optimization_barrier(fm2.reshape(F * V, E))
    fm1_flat = lax.optimization_barrier(fm1.reshape(F * V))

    e2, e1 = _get_sc_gather()(flat_idx, fm2_flat, fm1_flat)
    e2 = e2.reshape(B, F * E)
    e1 = e1.reshape(B, F)

    eye = jnp.eye(E, dtype=jnp.float32)
    rep = (jnp.arange(F)[:, None] ==
           (jnp.arange(F * E)[None, :] // E)).astype(jnp.float32)
    ssum = jnp.tile(eye, (F, 1))

    grid = (B // BB,)
    out = pl.pallas_call(
        _tc_body,
        grid=grid,
        in_specs=[
            pl.BlockSpec((BB, F * E), lambda i: (i, 0)),
            pl.BlockSpec((BB, F), lambda i: (i, 0)),
            pl.BlockSpec((BB, F), lambda i: (i, 0)),
            pl.BlockSpec((F, F * E), lambda i: (0, 0)),
            pl.BlockSpec((F * E, E), lambda i: (0, 0)),
            pl.BlockSpec((F * E, D1), lambda i: (0, 0)),
            pl.BlockSpec((1, D1), lambda i: (0, 0)),
            pl.BlockSpec((D1, D2), lambda i: (0, 0)),
            pl.BlockSpec((1, D2), lambda i: (0, 0)),
            pl.BlockSpec((1, 1), lambda i: (0, 0)),
        ],
        out_specs=pl.BlockSpec((BB,), lambda i: (i,)),
        out_shape=jax.ShapeDtypeStruct((B,), jnp.float32),
    )(e2, e1, Xv, rep, ssum, W1, b1.reshape(1, D1), W2, b2.reshape(1, D2),
      bias.reshape(1, 1))
    return out


def kernel(Xi, Xv, fm1, fm2, bias, W1, b1, W2, b2):
    return _deepfm(Xi, Xv, fm1, fm2, bias, W1, b1, W2, b2)


# speedup vs baseline: 1.0005x; 1.0001x over previous
import functools
import jax
import jax.numpy as jnp
from jax import lax
from jax.experimental import pallas as pl
from jax.experimental.pallas import tpu as pltpu
from jax.experimental.pallas import tpu_sc as plsc

F = 26
V = 100000
E = 16
B = 4096
D1 = 32
D2 = 32

NC = 2
NS = 16
NW = NC * NS
N = B * F
NPW = N // NW

CH = 128
NCHUNK = NPW // CH
NROW = N // CH


@functools.lru_cache(maxsize=None)
def _get_sc_gather():
    mesh = plsc.VectorSubcoreMesh(
        core_axis_name="c", subcore_axis_name="s",
        num_cores=NC, num_subcores=NS)

    @functools.partial(
        pl.kernel,
        out_type=(
            jax.ShapeDtypeStruct((NROW, CH, E), jnp.float32),
            jax.ShapeDtypeStruct((NROW, CH), jnp.float32),
        ),
        mesh=mesh,
        scratch_types=[
            pltpu.VMEM((NCHUNK, CH), jnp.int32),
            pltpu.VMEM((NCHUNK, CH, E), jnp.float32),
            pltpu.VMEM((NCHUNK, CH), jnp.float32),
            pltpu.SemaphoreType.DMA,
            pltpu.SemaphoreType.DMA,
        ],
        compiler_params=pltpu.CompilerParams(use_tc_tiling_on_sc=False),
    )
    def _sc_gather(idx_hbm, fm2_hbm, fm1_hbm, e2_out, e1_out,
                   idx_v, rows_v, r1_v, sem2, sem1):
        wid = lax.axis_index("s") * NC + lax.axis_index("c")
        base = wid * NCHUNK
        pltpu.sync_copy(idx_hbm.at[pl.ds(base, NCHUNK)], idx_v)
        cps = []
        for j in range(NCHUNK):
            cps.append(pltpu.async_copy(
                fm2_hbm.at[idx_v.at[j]], rows_v.at[j], sem2))
            cps.append(pltpu.async_copy(
                fm1_hbm.at[idx_v.at[j]], r1_v.at[j], sem1))
        for cp in cps:
            cp.wait()
        pltpu.sync_copy(rows_v, e2_out.at[pl.ds(base, NCHUNK)])
        pltpu.sync_copy(r1_v, e1_out.at[pl.ds(base, NCHUNK)])

    return _sc_gather


BB = 512


def _tc_body(e2_ref, e1_ref, xv_ref, rep_ref, ssum_ref, w1_ref, b1_ref,
             w2_ref, b2_ref, bias_ref, out_ref):
    e2 = e2_ref[...]
    xv = xv_ref[...]
    xvr = jnp.dot(xv, rep_ref[...], preferred_element_type=jnp.float32)
    e2v = e2 * xvr
    s = jnp.dot(e2v, ssum_ref[...], preferred_element_type=jnp.float32)
    sq = jnp.dot(e2v * e2v, ssum_ref[...], preferred_element_type=jnp.float32)
    second = 0.5 * jnp.sum(s * s - sq, axis=1)
    first = jnp.sum(e1_ref[...] * xv, axis=1)
    x1 = jnp.maximum(
        jnp.dot(e2v, w1_ref[...], preferred_element_type=jnp.float32)
        + b1_ref[...], 0.0)
    x2 = jnp.maximum(
        jnp.dot(x1, w2_ref[...], preferred_element_type=jnp.float32)
        + b2_ref[...], 0.0)
    out_ref[...] = first + second + jnp.sum(x2, axis=1) + bias_ref[0, 0]


@jax.jit
def _deepfm(Xi, Xv, fm1, fm2, bias, W1, b1, W2, b2):
    idx = Xi[:, :, 0]
    flat_idx = (idx + jnp.arange(F, dtype=jnp.int32)[None, :] * V).reshape(
        NROW, CH)
    fm2_flat = fm2.reshape(F * V, E)
    fm1_flat = fm1.reshape(F * V)

    e2, e1 = _get_sc_gather()(flat_idx, fm2_flat, fm1_flat)
    e2 = e2.reshape(B, F * E)
    e1 = e1.reshape(B, F)

    eye = jnp.eye(E, dtype=jnp.float32)
    rep = (jnp.arange(F)[:, None] ==
           (jnp.arange(F * E)[None, :] // E)).astype(jnp.float32)
    ssum = jnp.tile(eye, (F, 1))

    grid = (B // BB,)
    out = pl.pallas_call(
        _tc_body,
        grid=grid,
        in_specs=[
            pl.BlockSpec((BB, F * E), lambda i: (i, 0)),
            pl.BlockSpec((BB, F), lambda i: (i, 0)),
            pl.BlockSpec((BB, F), lambda i: (i, 0)),
            pl.BlockSpec((F, F * E), lambda i: (0, 0)),
            pl.BlockSpec((F * E, E), lambda i: (0, 0)),
            pl.BlockSpec((F * E, D1), lambda i: (0, 0)),
            pl.BlockSpec((1, D1), lambda i: (0, 0)),
            pl.BlockSpec((D1, D2), lambda i: (0, 0)),
            pl.BlockSpec((1, D2), lambda i: (0, 0)),
            pl.BlockSpec((1, 1), lambda i: (0, 0)),
        ],
        out_specs=pl.BlockSpec((BB,), lambda i: (i,)),
        out_shape=jax.ShapeDtypeStruct((B,), jnp.float32),
    )(e2, e1, Xv, rep, ssum, W1, b1.reshape(1, D1), W2, b2.reshape(1, D2),
      bias.reshape(1, 1))
    return out


def kernel(Xi, Xv, fm1, fm2, bias, W1, b1, W2, b2):
    return _deepfm(Xi, Xv, fm1, fm2, bias, W1, b1, W2, b2)
